# baseline (speedup 1.0000x reference)
import functools

import jax
import jax.numpy as jnp
from jax import lax
from jax.experimental import pallas as pl
from jax.experimental.pallas import tpu as pltpu
from jax.experimental.pallas import tpu_sc as plsc

B = 16384
D = 32
NC = 2
NS = 16
L = 16
NW = NC * NS
BPW = B // NW
CHUNK = 128
KCH = BPW // CHUNK
G = BPW // L

_mesh = plsc.VectorSubcoreMesh(
    core_axis_name="c", subcore_axis_name="s", num_cores=NC, num_subcores=NS
)


@functools.partial(
    pl.kernel,
    out_type=jax.ShapeDtypeStruct((B,), jnp.float32),
    mesh=_mesh,
    scratch_types=[
        pltpu.VMEM((KCH, CHUNK), jnp.int32),
        pltpu.VMEM((KCH, CHUNK), jnp.int32),
        pltpu.VMEM((BPW, D), jnp.float32),
        pltpu.VMEM((BPW, D), jnp.float32),
        pltpu.VMEM((BPW * L,), jnp.float32),
        pltpu.VMEM((BPW,), jnp.float32),
        pltpu.SemaphoreType.DMA,
        pltpu.SemaphoreType.DMA,
    ],
    compiler_params=pltpu.CompilerParams(
        needs_layout_passes=False, use_tc_tiling_on_sc=False
    ),
)
def _mf_sc(user_hbm, item_hbm, utab_hbm, itab_hbm, out_hbm,
           uidx_v, iidx_v, urows_v, irows_v, psum_v, out_v, sem_u, sem_i):
    wid = lax.axis_index("s") * NC + lax.axis_index("c")
    base = wid * BPW

    pltpu.sync_copy(user_hbm.at[wid], uidx_v)
    pltpu.sync_copy(item_hbm.at[wid], iidx_v)

    copies = []
    for k in range(KCH):
        dst = pl.ds(k * CHUNK, CHUNK)
        copies.append(
            pltpu.async_copy(utab_hbm.at[uidx_v.at[k]], urows_v.at[dst], sem_u)
        )
        copies.append(
            pltpu.async_copy(itab_hbm.at[iidx_v.at[k]], irows_v.at[dst], sem_i)
        )
    for cp in copies:
        cp.wait()

    def prow(b, carry):
        u0 = urows_v[b, pl.ds(0, L)]
        u1 = urows_v[b, pl.ds(L, L)]
        v0 = irows_v[b, pl.ds(0, L)]
        v1 = irows_v[b, pl.ds(L, L)]
        psum_v[pl.ds(b * L, L)] = u0 * v0 + u1 * v1
        return carry

    lax.fori_loop(0, BPW, prow, 0)

    lane = lax.iota(jnp.int32, L)

    def group(g, carry):
        rowbase = (g * L + lane) * L
        acc = plsc.load_gather(psum_v, [rowbase])
        for l in range(1, L):
            acc = acc + plsc.load_gather(psum_v, [rowbase + l])
        out_v[pl.ds(g * L, L)] = acc
        return carry

    lax.fori_loop(0, G, group, 0)

    pltpu.sync_copy(out_v, out_hbm.at[pl.ds(base, BPW)])


def kernel(user, item, user_table, item_table):
    user_r = user.astype(jnp.int32).reshape(NW, KCH, CHUNK)
    item_r = item.astype(jnp.int32).reshape(NW, KCH, CHUNK)
    return _mf_sc(user_r, item_r, user_table, item_table)
